# Initial kernel scaffold; baseline (speedup 1.0000x reference)
import jax
import jax.numpy as jnp
from jax.experimental import pallas as pl


def kernel(x, edge_index, W1, b1, W2, b2):
    raise NotImplementedError("write your pallas kernel here")


# speedup vs baseline: 29.2296x; 29.2296x over previous
import functools

import jax
import jax.numpy as jnp
from jax import lax
from jax.experimental import pallas as pl
from jax.experimental.pallas import tpu as pltpu
from jax.experimental.pallas import tpu_sc as plsc

N = 10000
E = 320000
D_IN = 128
D_HID = 16
D_OUT = 40

NACC = 10240
CW = 128
CH = 80
NC = 2
NS = 16
NW = NC * NS
EPAD = NW * CH * CW

_mesh = plsc.VectorSubcoreMesh(core_axis_name="c", subcore_axis_name="s")
_sc_params = pltpu.CompilerParams(use_tc_tiling_on_sc=False)


@functools.partial(
    pl.kernel,
    out_type=jax.ShapeDtypeStruct((NC, NACC), jnp.float32),
    mesh=_mesh,
    scratch_types=[
        pltpu.VMEM((CH, CW), jnp.int32),
        pltpu.VMEM((CW,), jnp.float32),
        pltpu.VMEM_SHARED((NACC,), jnp.float32),
        pltpu.SemaphoreType.DMA,
    ],
    compiler_params=_sc_params,
)
def _deg_kernel(dst_hbm, zeros_hbm, ones_hbm, out_hbm, dst_v, ones_v, acc_sh, sem):
    cid = lax.axis_index("c")
    sid = lax.axis_index("s")
    wid = sid * NC + cid

    @pl.when(sid == 0)
    def _init():
        pltpu.sync_copy(zeros_hbm, acc_sh)

    pltpu.sync_copy(ones_hbm, ones_v)
    pltpu.sync_copy(dst_hbm.at[pl.ds(wid * CH, CH)], dst_v)
    plsc.subcore_barrier()

    def body(j, carry):
        pltpu.sync_copy(ones_v, acc_sh.at[dst_v.at[j]], add=True)
        return carry

    lax.fori_loop(0, CH, body, 0)
    plsc.subcore_barrier()

    @pl.when(sid == 0)
    def _out():
        pltpu.sync_copy(acc_sh, out_hbm.at[cid])


@functools.partial(
    pl.kernel,
    out_type=jax.ShapeDtypeStruct((NC, NACC, D_HID), jnp.float32),
    mesh=_mesh,
    scratch_types=[
        pltpu.VMEM((CH, CW), jnp.int32),
        pltpu.VMEM((CH, CW), jnp.int32),
        pltpu.VMEM((CW, D_HID), jnp.float32),
        pltpu.VMEM_SHARED((NACC, D_HID), jnp.float32),
        pltpu.SemaphoreType.DMA,
    ],
    compiler_params=_sc_params,
)
def _agg_kernel(h_hbm, src_hbm, dst_hbm, zeros_hbm, out_hbm,
                src_v, dst_v, rows_v, acc_sh, sem):
    cid = lax.axis_index("c")
    sid = lax.axis_index("s")
    wid = sid * NC + cid

    @pl.when(sid == 0)
    def _init():
        pltpu.sync_copy(zeros_hbm, acc_sh)

    pltpu.sync_copy(src_hbm.at[pl.ds(wid * CH, CH)], src_v)
    pltpu.sync_copy(dst_hbm.at[pl.ds(wid * CH, CH)], dst_v)
    plsc.subcore_barrier()

    def body(j, carry):
        pltpu.async_copy(h_hbm.at[src_v.at[j]], rows_v, sem).wait()
        pltpu.sync_copy(rows_v, acc_sh.at[dst_v.at[j]], add=True)
        return carry

    lax.fori_loop(0, CH, body, 0)
    plsc.subcore_barrier()

    @pl.when(sid == 0)
    def _out():
        pltpu.sync_copy(acc_sh, out_hbm.at[cid])


_RB = 1024


def _dis(degp_ref):
    return lax.rsqrt(degp_ref[0, :] + degp_ref[1, :] + 1.0)[:, None]


def _tc1_body(x_ref, w_ref, degp_ref, o_ref):
    h = jnp.dot(x_ref[...], w_ref[...], preferred_element_type=jnp.float32)
    o_ref[...] = h * _dis(degp_ref)


def _tc2_body(aggp_ref, hp_ref, degp_ref, b_ref, o_ref):
    dis = _dis(degp_ref)
    pre = dis * (aggp_ref[0] + aggp_ref[1] + hp_ref[...]) + b_ref[...]
    o_ref[...] = dis * jnp.maximum(pre, 0.0)


def _tc3_body(aggp_ref, hq_ref, degp_ref, w_ref, b_ref, o_ref):
    z = _dis(degp_ref) * (aggp_ref[0] + aggp_ref[1] + hq_ref[...])
    o_ref[...] = jnp.dot(z, w_ref[...], preferred_element_type=jnp.float32) + b_ref[...]


def _tc1(x_pad, W1, degp):
    return pl.pallas_call(
        _tc1_body,
        grid=(NACC // _RB,),
        in_specs=[
            pl.BlockSpec((_RB, D_IN), lambda i: (i, 0)),
            pl.BlockSpec((D_IN, D_HID), lambda i: (0, 0)),
            pl.BlockSpec((NC, _RB), lambda i: (0, i)),
        ],
        out_specs=pl.BlockSpec((_RB, D_HID), lambda i: (i, 0)),
        out_shape=jax.ShapeDtypeStruct((NACC, D_HID), jnp.float32),
    )(x_pad, W1, degp)


def _tc2(aggp, hp, degp, b1):
    return pl.pallas_call(
        _tc2_body,
        grid=(NACC // _RB,),
        in_specs=[
            pl.BlockSpec((NC, _RB, D_HID), lambda i: (0, i, 0)),
            pl.BlockSpec((_RB, D_HID), lambda i: (i, 0)),
            pl.BlockSpec((NC, _RB), lambda i: (0, i)),
            pl.BlockSpec((1, D_HID), lambda i: (0, 0)),
        ],
        out_specs=pl.BlockSpec((_RB, D_HID), lambda i: (i, 0)),
        out_shape=jax.ShapeDtypeStruct((NACC, D_HID), jnp.float32),
    )(aggp, hp, degp, b1)


def _tc3(aggp, hq, degp, W2, b2):
    return pl.pallas_call(
        _tc3_body,
        grid=(NACC // _RB,),
        in_specs=[
            pl.BlockSpec((NC, _RB, D_HID), lambda i: (0, i, 0)),
            pl.BlockSpec((_RB, D_HID), lambda i: (i, 0)),
            pl.BlockSpec((NC, _RB), lambda i: (0, i)),
            pl.BlockSpec((D_HID, D_OUT), lambda i: (0, 0)),
            pl.BlockSpec((1, D_OUT), lambda i: (0, 0)),
        ],
        out_specs=pl.BlockSpec((_RB, D_OUT), lambda i: (i, 0)),
        out_shape=jax.ShapeDtypeStruct((NACC, D_OUT), jnp.float32),
    )(aggp, hq, degp, W2, b2)


def kernel(x, edge_index, W1, b1, W2, b2):
    ei = edge_index.astype(jnp.int32)
    pad = jnp.full((EPAD - E,), N, jnp.int32)
    src_r = jnp.concatenate([ei[0], pad]).reshape(NW * CH, CW)
    dst_r = jnp.concatenate([ei[1], pad]).reshape(NW * CH, CW)
    x_pad = jnp.pad(x, ((0, NACC - N), (0, 0)))
    zeros1 = jnp.zeros((NACC,), jnp.float32)
    zeros16 = jnp.zeros((NACC, D_HID), jnp.float32)
    ones_row = jnp.ones((CW,), jnp.float32)

    degp = _deg_kernel(dst_r, zeros1, ones_row)
    hp = _tc1(x_pad, W1, degp)
    aggp1 = _agg_kernel(hp, src_r, dst_r, zeros16)
    hq = _tc2(aggp1, hp, degp, b1.reshape(1, D_HID))
    aggp2 = _agg_kernel(hq, src_r, dst_r, zeros16)
    out = _tc3(aggp2, hq, degp, W2, b2.reshape(1, D_OUT))
    return out[:N]
